# baseline (speedup 1.0000x reference)
import functools

import jax
import jax.numpy as jnp
from jax import lax
from jax.experimental import pallas as pl
from jax.experimental.pallas import tpu as pltpu
from jax.experimental.pallas import tpu_sc as plsc

_NUM_FIELDS = 26
_VOCAB = 100000
_EMBED_DIM = 16
_NUM_DENSE = 13

_NUM_CORES = 2
_NUM_SUBCORES = 16
_NUM_WORKERS = _NUM_CORES * _NUM_SUBCORES

_GATHER = 128
_GPS = 13
_SUPER = _GATHER * _GPS


@functools.partial(jax.jit, static_argnames=("batch",))
def _embed_flat(idx_flat, tables_flat, *, batch):
    total = batch * _NUM_FIELDS
    per_w = total // _NUM_WORKERS
    n_super = per_w // _SUPER
    mesh = plsc.VectorSubcoreMesh(
        core_axis_name="c", subcore_axis_name="s")

    @functools.partial(
        pl.kernel,
        out_type=jax.ShapeDtypeStruct((total, _EMBED_DIM), jnp.float32),
        mesh=mesh,
        compiler_params=pltpu.CompilerParams(use_tc_tiling_on_sc=False),
        scratch_types=[
            pltpu.VMEM((per_w,), jnp.int32),
            pltpu.VMEM((_SUPER, _EMBED_DIM), jnp.float32),
            pltpu.VMEM((_SUPER, _EMBED_DIM), jnp.float32),
            pltpu.SemaphoreType.DMA,
            pltpu.SemaphoreType.DMA,
            pltpu.SemaphoreType.DMA,
            pltpu.SemaphoreType.DMA,
        ],
    )
    def k(idx_hbm, tab_hbm, out_hbm, idx_v, buf_a, buf_b, sem_i, sem_g,
          sem_oa, sem_ob):
        wid = lax.axis_index("s") * _NUM_CORES + lax.axis_index("c")
        wbase = wid * per_w

        pltpu.async_copy(
            idx_hbm.at[pl.ds(wbase, per_w)], idx_v, sem_i,
        ).wait()

        bufs = (buf_a, buf_b)
        out_sems = (sem_oa, sem_ob)

        def fire_gathers(s, buf):
            sbase = s * _SUPER
            return [
                pltpu.async_copy(
                    tab_hbm.at[idx_v.at[pl.ds(sbase + j * _GATHER,
                                              _GATHER)]],
                    buf.at[pl.ds(j * _GATHER, _GATHER)],
                    sem_g,
                )
                for j in range(_GPS)
            ]

        gather_cps = fire_gathers(0, bufs[0])
        out_cps = [None, None]
        for s in range(n_super):
            cur = s % 2
            nxt = (s + 1) % 2
            for cp in gather_cps:
                cp.wait()
            if s + 1 < n_super:
                if out_cps[nxt] is not None:
                    out_cps[nxt].wait()
                gather_cps = fire_gathers(s + 1, bufs[nxt])
            out_cps[cur] = pltpu.async_copy(
                bufs[cur],
                out_hbm.at[pl.ds(wbase + s * _SUPER, _SUPER)],
                out_sems[cur],
            )
        for cp in out_cps:
            if cp is not None:
                cp.wait()

    return k(idx_flat, tables_flat)


def kernel(categorical, numerical, tables):
    batch = categorical.shape[0]
    idx_flat = (categorical.astype(jnp.int32)
                + (jnp.arange(_NUM_FIELDS, dtype=jnp.int32)
                   * _VOCAB)[None, :]).reshape(batch * _NUM_FIELDS)
    tables_flat = tables.reshape(_NUM_FIELDS * _VOCAB, _EMBED_DIM)
    emb = _embed_flat(idx_flat, tables_flat, batch=batch)
    emb_bf = emb.reshape(batch, _NUM_FIELDS * _EMBED_DIM)
    return jnp.concatenate([numerical, emb_bf], axis=-1)


# speedup vs baseline: 2.6113x; 2.6113x over previous
import functools

import jax
import jax.numpy as jnp
from jax import lax
from jax.experimental import pallas as pl
from jax.experimental.pallas import tpu as pltpu
from jax.experimental.pallas import tpu_sc as plsc

_NUM_FIELDS = 26
_VOCAB = 100000
_EMBED_DIM = 16
_NUM_DENSE = 13

_NUM_CORES = 2
_NUM_SUBCORES = 16
_NUM_WORKERS = _NUM_CORES * _NUM_SUBCORES

_ROWS = _NUM_FIELDS * _EMBED_DIM
_ROWS_PER_W = _ROWS // _NUM_WORKERS
_LANES = 16
_UNROLL = 4


@functools.partial(jax.jit, static_argnames=("batch",))
def _embed_cols(idx_t, tab2d, num_t, *, batch):
    half = batch // 2
    mesh = plsc.VectorSubcoreMesh(
        core_axis_name="c", subcore_axis_name="s")

    @functools.partial(
        pl.kernel,
        out_type=jax.ShapeDtypeStruct((_NUM_DENSE + _ROWS, batch),
                                      jnp.float32),
        mesh=mesh,
        compiler_params=pltpu.CompilerParams(
            use_tc_tiling_on_sc=False, needs_layout_passes=False),
        scratch_types=[
            pltpu.VMEM((_VOCAB,), jnp.float32),
            pltpu.VMEM((half,), jnp.int32),
            pltpu.VMEM((half,), jnp.float32),
            pltpu.SemaphoreType.DMA,
            pltpu.SemaphoreType.DMA,
            pltpu.SemaphoreType.DMA,
        ],
    )
    def k(idx_hbm, tab_hbm, num_hbm, out_hbm, trow, idxb, obuf,
          sem_t, sem_i, sem_o):
        wid = lax.axis_index("s") * _NUM_CORES + lax.axis_index("c")

        def gather_half(h, p):
            pltpu.async_copy(
                idx_hbm.at[p // _EMBED_DIM, pl.ds(h * half, half)],
                idxb, sem_i,
            ).wait()

            def body(k0, _):
                base = k0 * (_LANES * _UNROLL)
                for u in range(_UNROLL):
                    off = base + u * _LANES
                    iv = idxb[pl.ds(off, _LANES)]
                    vals = plsc.load_gather(trow, [iv])
                    obuf[pl.ds(off, _LANES)] = vals
                return 0

            lax.fori_loop(0, half // (_LANES * _UNROLL), body, 0)
            pltpu.async_copy(
                obuf,
                out_hbm.at[_NUM_DENSE + p, pl.ds(h * half, half)],
                sem_o,
            ).wait()

        for j in range(_ROWS_PER_W):
            p = wid * _ROWS_PER_W + j
            pltpu.async_copy(tab_hbm.at[p], trow, sem_t).wait()
            gather_half(0, p)
            gather_half(1, p)

        @pl.when(wid < _NUM_DENSE)
        def _():
            for h in range(2):
                pltpu.async_copy(
                    num_hbm.at[wid, pl.ds(h * half, half)], obuf, sem_i,
                ).wait()
                pltpu.async_copy(
                    obuf, out_hbm.at[wid, pl.ds(h * half, half)], sem_o,
                ).wait()

    return k(idx_t, tab2d, num_t)


def kernel(categorical, numerical, tables):
    batch = categorical.shape[0]
    idx_t = jnp.swapaxes(categorical.astype(jnp.int32), 0, 1)
    tab2d = jnp.swapaxes(tables, 1, 2).reshape(_ROWS, _VOCAB)
    num_t = jnp.swapaxes(numerical, 0, 1)
    out_t = _embed_cols(idx_t, tab2d, num_t, batch=batch)
    return jnp.swapaxes(out_t, 0, 1)
